# baseline (speedup 1.0000x reference)
import jax
import jax.numpy as jnp
from jax.experimental import pallas as pl
from jax.experimental.pallas import tpu as pltpu

_N = 4096
_D = 128
_BM = 512
_BK = 1024
_HK = _N // 2
_NI = _N // _BM
_NK = _HK // _BK


def _body(x_ref, wi_ref, ws_ref, l0_ref, l1_ref, u0_ref, u1_ref, out_ref,
          acc_ref, hi_ref, hs_ref):
    i = pl.program_id(0)
    k = pl.program_id(1)

    @pl.when(i == 0)
    def _():
        for half in (0, 1):
            off = half * _HK + k * _BK
            xb = x_ref[pl.ds(off, _BK), :]
            hi_ref[pl.ds(off, _BK), :] = jnp.dot(
                xb, wi_ref[...], preferred_element_type=jnp.float32)
            hs_ref[pl.ds(off, _BK), :] = jnp.dot(
                xb, ws_ref[...], preferred_element_type=jnp.float32)

    @pl.when(k == 0)
    def _():
        acc_ref[...] = jnp.zeros_like(acc_ref)

    hi0 = hi_ref[pl.ds(k * _BK, _BK), :]
    hi1 = hi_ref[pl.ds(_HK + k * _BK, _BK), :]
    hs0 = hs_ref[pl.ds(k * _BK, _BK), :]
    hs1 = hs_ref[pl.ds(_HK + k * _BK, _BK), :]
    acc_ref[...] += (
        jnp.dot(l0_ref[...], hi0, preferred_element_type=jnp.float32)
        + jnp.dot(l1_ref[...], hi1, preferred_element_type=jnp.float32)
        + jnp.dot(u0_ref[...], hs0, preferred_element_type=jnp.float32)
        + jnp.dot(u1_ref[...], hs1, preferred_element_type=jnp.float32))

    @pl.when(k == _NK - 1)
    def _():
        out_ref[...] = jnp.maximum(acc_ref[...], 0.0)


def kernel(x, lower_neighborhood, upper_neighborhood, W_irr, W_sol):
    nhalf = _HK // _BK
    return pl.pallas_call(
        _body,
        grid=(_NI, _NK),
        in_specs=[
            pl.BlockSpec((_N, _D), lambda i, k: (0, 0)),
            pl.BlockSpec((_D, _D), lambda i, k: (0, 0)),
            pl.BlockSpec((_D, _D), lambda i, k: (0, 0)),
            pl.BlockSpec((_BM, _BK), lambda i, k: (i, k)),
            pl.BlockSpec((_BM, _BK), lambda i, k: (i, 2 + k)),
            pl.BlockSpec((_BM, _BK), lambda i, k: (i, k)),
            pl.BlockSpec((_BM, _BK), lambda i, k: (i, 2 + k)),
        ],
        out_specs=pl.BlockSpec((_BM, _D), lambda i, k: (i, 0)),
        out_shape=jax.ShapeDtypeStruct((_N, _D), jnp.float32),
        scratch_shapes=[
            pltpu.VMEM((_BM, _D), jnp.float32),
            pltpu.VMEM((_N, _D), jnp.float32),
            pltpu.VMEM((_N, _D), jnp.float32),
        ],
        compiler_params=pltpu.CompilerParams(
            dimension_semantics=("arbitrary", "arbitrary")),
    )(x, W_irr, W_sol,
      lower_neighborhood, lower_neighborhood,
      upper_neighborhood, upper_neighborhood)
